# baseline (speedup 1.0000x reference)
import functools

import jax
import jax.numpy as jnp
from jax import lax
from jax.experimental import pallas as pl
from jax.experimental.pallas import tpu as pltpu
from jax.experimental.pallas import tpu_sc as plsc

_NC = 8
_C = 2048
_H = _C // 2
_N = 8192
_BLK = 256
_GRID = _N // _BLK

_L = 16
_NW = 32
_RPW = _N // _NW
_CH = 16
_CHM = 128
_W = 128


def _bce(z, y):
    return jnp.maximum(z, 0.0) - z * y + jnp.log1p(jnp.exp(-jnp.abs(z)))


def _head0_kernel(x_ref, labels_ref, w1_ref, b1_ref, w2_ref, b2_ref,
                  cp_ref, side_ref, mask_ref, pn_ref, acc_pn):
    i = pl.program_id(0)

    @pl.when(i == 0)
    def _init():
        acc_pn[:, :] = jnp.zeros_like(acc_pn)

    xb = x_ref[:, :].astype(jnp.bfloat16)
    h = jax.lax.dot_general(xb, w1_ref[:, :], (((1,), (1,)), ((), ())),
                            preferred_element_type=jnp.float32)
    h = jnp.maximum(h + b1_ref[:, :], 0.0)
    z = jax.lax.dot_general(h.astype(jnp.bfloat16), w2_ref[:, :],
                            (((1,), (0,)), ((), ())),
                            preferred_element_type=jnp.float32)
    z = z + b2_ref[0, 0]
    y0 = labels_ref[:, 0:1]
    acc_pn[:, :] += jnp.sum(_bce(z, y0), axis=(0, 1), keepdims=True)
    probs = jax.nn.sigmoid(z)
    mask = probs >= 0.5
    cp_ref[:, :] = jnp.concatenate(
        [probs, jnp.zeros((_BLK, _W - 1), jnp.float32)], axis=1)
    side_ref[:, :] = jnp.concatenate(
        [labels_ref[:, 1:8], probs, jnp.zeros((_BLK, _W - 8), jnp.float32)],
        axis=1)
    mask_ref[:, :] = mask.astype(jnp.int32)

    @pl.when(i == _GRID - 1)
    def _fin():
        pn_ref[:, :] = acc_pn[:, :] * (1.0 / _N)


def _run_head0(x, labels, w1b, b1, w2b, b2):
    full = lambda *shape: pl.BlockSpec(shape, lambda i: (0,) * len(shape))
    return pl.pallas_call(
        _head0_kernel,
        grid=(_GRID,),
        in_specs=[
            pl.BlockSpec((_BLK, _C), lambda i: (i, 0)),
            pl.BlockSpec((_BLK, _NC), lambda i: (i, 0)),
            full(_H, _C),
            full(1, _H),
            full(_H, 1),
            full(1, 1),
        ],
        out_specs=[
            pl.BlockSpec((_BLK, _W), lambda i: (i, 0)),
            pl.BlockSpec((_BLK, _W), lambda i: (i, 0)),
            pl.BlockSpec((_BLK, 1), lambda i: (i, 0)),
            pl.BlockSpec((1, 1), lambda i: (0, 0)),
        ],
        out_shape=(
            jax.ShapeDtypeStruct((_N, _W), jnp.float32),
            jax.ShapeDtypeStruct((_N, _W), jnp.float32),
            jax.ShapeDtypeStruct((_N, 1), jnp.int32),
            jax.ShapeDtypeStruct((1, 1), jnp.float32),
        ),
        scratch_shapes=[pltpu.VMEM((1, 1), jnp.float32)],
        compiler_params=pltpu.CompilerParams(
            dimension_semantics=("arbitrary",),
            vmem_limit_bytes=64 * 1024 * 1024,
        ),
    )(x, labels, w1b, b1, w2b, b2)


_GDN = lax.GatherDimensionNumbers(offset_dims=(), collapsed_slice_dims=(0,),
                                  start_index_map=(0,))


def _take(v, idx):
    return lax.gather(v, idx[:, None], _GDN, (1,),
                      mode=lax.GatherScatterMode.PROMISE_IN_BOUNDS)


def _lane():
    return lax.iota(jnp.int32, _L)


def _prefix_incl(v):
    lane = _lane()
    for k in (1, 2, 4, 8):
        idx = jnp.maximum(lane - k, 0)
        msk = jnp.minimum(jnp.maximum(lane - (k - 1), 0), 1)
        v = v + _take(v, idx) * msk
    return v


def _splat_last(v):
    return _take(v, _lane() * 0 + (_L - 1))


_SC_MESH = plsc.VectorSubcoreMesh(core_axis_name="c", subcore_axis_name="s")


@functools.partial(
    pl.kernel, mesh=_SC_MESH,
    out_type=[jax.ShapeDtypeStruct((_N, _C), jnp.float32),
              jax.ShapeDtypeStruct((_N, _W), jnp.float32),
              jax.ShapeDtypeStruct((16,), jnp.int32),
              jax.ShapeDtypeStruct((_N,), jnp.int32),
              jax.ShapeDtypeStruct((_N + _L,), jnp.int32)],
    scratch_types=[pltpu.VMEM((_N,), jnp.int32),
                   pltpu.VMEM((_N,), jnp.int32),
                   pltpu.VMEM((_N,), jnp.int32),
                   pltpu.VMEM((_N,), jnp.int32),
                   pltpu.VMEM((_CH,), jnp.int32),
                   pltpu.VMEM((_CH, _C), jnp.float32),
                   pltpu.VMEM((_CH, _W), jnp.float32),
                   pltpu.VMEM((16,), jnp.int32),
                   pltpu.SemaphoreType.DMA,
                   pltpu.SemaphoreType.DMA])
def _sc_gather(mask_hbm, x_hbm, side_hbm,
               xg_hbm, sideg_hbm, cnt_hbm, sel_hbm, idx_hbm,
               mask_v, vals_v, addr_v, sel_v, idxw_v, rows_v, srow_v, cnt_v,
               sem, sem2):
    wid = lax.axis_index("s") * 2 + lax.axis_index("c")
    pltpu.sync_copy(mask_hbm, mask_v)
    lane = _lane()

    def comp(i, wpos_v):
        mi = mask_v[pl.ds(i * _L, _L)]
        pre = _prefix_incl(mi)
        tot = _splat_last(pre)
        gpos = wpos_v + pre - mi
        src = lane + i * _L
        addr = gpos * mi + (_N + lane) * (1 - mi)
        vals_v[pl.ds(i * _L, _L)] = src
        addr_v[pl.ds(i * _L, _L)] = addr
        sel_v[pl.ds(i * _L, _L)] = gpos * mi + (_N + src) * (1 - mi)
        return wpos_v + tot

    wpos_v = lax.fori_loop(0, _N // _L, comp, jnp.zeros((_L,), jnp.int32))

    pltpu.async_copy(vals_v, idx_hbm.at[addr_v], sem2).wait()

    @pl.when(wid == 0)
    def _():
        cnt_v[:] = wpos_v
        pltpu.sync_copy(cnt_v, cnt_hbm)
        pltpu.sync_copy(sel_v, sel_hbm)

    for c in range(_RPW // _CH):
        base = wid * _RPW + c * _CH
        pltpu.sync_copy(idx_hbm.at[pl.ds(base, _CH)], idxw_v)
        iv = idxw_v[pl.ds(0, _CH)]
        idxw_v[pl.ds(0, _CH)] = jnp.minimum(jnp.maximum(iv, 0), _N - 1)
        pltpu.async_copy(x_hbm.at[idxw_v], rows_v, sem).wait()
        pltpu.sync_copy(rows_v, xg_hbm.at[pl.ds(base, _CH)])
        pltpu.async_copy(side_hbm.at[idxw_v], srow_v, sem).wait()
        pltpu.sync_copy(srow_v, sideg_hbm.at[pl.ds(base, _CH)])


def _heads_kernel(cnt_ref, xg_ref, sideg_ref, w1_ref, b1_ref, w2_ref, b2_ref,
                  zg_ref, cls_ref, acc_cls):
    j = pl.program_id(0)
    cnt = cnt_ref[0]

    @pl.when(j == 0)
    def _init():
        acc_cls[:, :] = jnp.zeros_like(acc_cls)

    @pl.when(j * _BLK < cnt)
    def _active():
        xb = xg_ref[:, :].astype(jnp.bfloat16)
        rowid = j * _BLK + lax.broadcasted_iota(jnp.int32, (_BLK, 1), 0)
        lmask = (rowid < cnt).astype(jnp.float32)
        hbs = []
        for c in range(_NC - 1):
            h = jax.lax.dot_general(xb, w1_ref[c], (((1,), (1,)), ((), ())),
                                    preferred_element_type=jnp.float32)
            h = jnp.maximum(h + b1_ref[c], 0.0)
            hbs.append(h.astype(jnp.bfloat16))
        cols = [sideg_ref[:, 7:8]]
        part = jnp.zeros((1, 1), jnp.float32)
        for c in range(_NC - 1):
            z = jax.lax.dot_general(hbs[c], w2_ref[c], (((1,), (0,)), ((), ())),
                                    preferred_element_type=jnp.float32)
            z = z + b2_ref[c, 0]
            y = sideg_ref[:, c:c + 1]
            part = part + jnp.sum(_bce(z, y) * lmask, axis=(0, 1), keepdims=True)
            cols.append(jax.nn.sigmoid(z))
        acc_cls[:, :] += part
        zg_ref[:, :] = jnp.concatenate(
            cols + [jnp.zeros((_BLK, _W - 8), jnp.float32)], axis=1)

    @pl.when(j == _GRID - 1)
    def _fin():
        cls_ref[:, :] = acc_cls[:, :] / jnp.maximum(cnt.astype(jnp.float32), 1.0)


def _run_heads(cnt, xg, sideg, w1, b1, w2, b2):
    full = lambda *shape: pl.BlockSpec(shape, lambda i, c: (0,) * len(shape))
    grid_spec = pltpu.PrefetchScalarGridSpec(
        num_scalar_prefetch=1,
        grid=(_GRID,),
        in_specs=[
            pl.BlockSpec((_BLK, _C), lambda i, c: (i, 0)),
            pl.BlockSpec((_BLK, _W), lambda i, c: (i, 0)),
            full(_NC - 1, _H, _C),
            full(_NC - 1, 1, _H),
            full(_NC - 1, _H, 1),
            full(_NC - 1, 1),
        ],
        out_specs=[
            pl.BlockSpec((_BLK, _W), lambda i, c: (i, 0)),
            pl.BlockSpec((1, 1), lambda i, c: (0, 0)),
        ],
        scratch_shapes=[pltpu.VMEM((1, 1), jnp.float32)],
    )
    return pl.pallas_call(
        _heads_kernel,
        grid_spec=grid_spec,
        out_shape=(
            jax.ShapeDtypeStruct((_N, _W), jnp.float32),
            jax.ShapeDtypeStruct((1, 1), jnp.float32),
        ),
        compiler_params=pltpu.CompilerParams(
            dimension_semantics=("arbitrary",),
            vmem_limit_bytes=64 * 1024 * 1024,
        ),
    )(cnt, xg, sideg, w1, b1, w2, b2)


@functools.partial(
    pl.kernel, mesh=_SC_MESH,
    out_type=jax.ShapeDtypeStruct((_N, _W), jnp.float32),
    scratch_types=[pltpu.VMEM((_CHM,), jnp.int32),
                   pltpu.VMEM((_CHM, _W), jnp.float32),
                   pltpu.SemaphoreType.DMA])
def _sc_merge(sel_hbm, src_hbm, out_hbm, selw_v, rows_v, sem):
    wid = lax.axis_index("s") * 2 + lax.axis_index("c")
    for c in range(_RPW // _CHM):
        base = wid * _RPW + c * _CHM
        pltpu.sync_copy(sel_hbm.at[pl.ds(base, _CHM)], selw_v)
        pltpu.async_copy(src_hbm.at[selw_v], rows_v, sem).wait()
        pltpu.sync_copy(rows_v, out_hbm.at[pl.ds(base, _CHM)])


def kernel(fuse_embeddings, segment_classes, pn_W1, pn_b1, pn_W2, pn_b2,
           cat_W1, cat_b1, cat_W2, cat_b2):
    seg = segment_classes.reshape(-1).astype(jnp.int32)
    lab0 = (seg > 0).astype(jnp.float32)
    labc = (seg[:, None] == jnp.arange(1, _NC, dtype=jnp.int32)[None, :]).astype(jnp.float32)
    labels = jnp.concatenate([lab0[:, None], labc], axis=1)

    x = fuse_embeddings.reshape(_N, _C)

    cp16, side, maskN1, pn = _run_head0(
        x, labels, pn_W1.astype(jnp.bfloat16), pn_b1.reshape(1, _H),
        pn_W2.astype(jnp.bfloat16).reshape(_H, 1), pn_b2.reshape(1, 1))

    xg, sideg, cnt16, sel, _ = _sc_gather(maskN1.reshape(_N), x, side)

    zg, cls = _run_heads(
        cnt16[0:1], xg, sideg,
        cat_W1.astype(jnp.bfloat16),
        cat_b1.reshape(_NC - 1, 1, _H),
        cat_W2.astype(jnp.bfloat16).reshape(_NC - 1, _H, 1),
        cat_b2.reshape(_NC - 1, 1))

    src = jnp.concatenate([zg, cp16], axis=0)
    cpf = _sc_merge(sel, src)

    return pn[0, 0], cls.reshape(1), cpf[:, :8]


# speedup vs baseline: 63.2719x; 63.2719x over previous
import functools

import jax
import jax.numpy as jnp
from jax import lax
from jax.experimental import pallas as pl
from jax.experimental.pallas import tpu as pltpu
from jax.experimental.pallas import tpu_sc as plsc

_NC = 8
_C = 2048
_H = _C // 2
_N = 8192
_BLK = 256
_GRID = _N // _BLK

_L = 16
_NW = 32
_RPW = _N // _NW
_CH = 16
_CHM = 128
_W = 128


def _bce(z, y):
    return jnp.maximum(z, 0.0) - z * y + jnp.log1p(jnp.exp(-jnp.abs(z)))


def _head0_kernel(x_ref, labels_ref, w1_ref, b1_ref, w2_ref, b2_ref,
                  cp_ref, side_ref, mask_ref, pn_ref, acc_pn):
    i = pl.program_id(0)

    @pl.when(i == 0)
    def _init():
        acc_pn[:, :] = jnp.zeros_like(acc_pn)

    xb = x_ref[:, :].astype(jnp.bfloat16)
    h = jax.lax.dot_general(xb, w1_ref[:, :], (((1,), (1,)), ((), ())),
                            preferred_element_type=jnp.float32)
    h = jnp.maximum(h + b1_ref[:, :], 0.0)
    z = jax.lax.dot_general(h.astype(jnp.bfloat16), w2_ref[:, :],
                            (((1,), (0,)), ((), ())),
                            preferred_element_type=jnp.float32)
    z = z + b2_ref[0, 0]
    y0 = labels_ref[:, 0:1]
    acc_pn[:, :] += jnp.sum(_bce(z, y0), axis=(0, 1), keepdims=True)
    probs = jax.nn.sigmoid(z)
    mask = probs >= 0.5
    cp_ref[:, :] = jnp.concatenate(
        [probs, jnp.zeros((_BLK, _W - 1), jnp.float32)], axis=1)
    side_ref[:, :] = jnp.concatenate(
        [labels_ref[:, 1:8], probs, jnp.zeros((_BLK, _W - 8), jnp.float32)],
        axis=1)
    mask_ref[:, :] = mask.astype(jnp.int32)

    @pl.when(i == _GRID - 1)
    def _fin():
        pn_ref[:, :] = acc_pn[:, :] * (1.0 / _N)


def _run_head0(x, labels, w1b, b1, w2b, b2):
    full = lambda *shape: pl.BlockSpec(shape, lambda i: (0,) * len(shape))
    return pl.pallas_call(
        _head0_kernel,
        grid=(_GRID,),
        in_specs=[
            pl.BlockSpec((_BLK, _C), lambda i: (i, 0)),
            pl.BlockSpec((_BLK, _NC), lambda i: (i, 0)),
            full(_H, _C),
            full(1, _H),
            full(_H, 1),
            full(1, 1),
        ],
        out_specs=[
            pl.BlockSpec((_BLK, _W), lambda i: (i, 0)),
            pl.BlockSpec((_BLK, _W), lambda i: (i, 0)),
            pl.BlockSpec((_BLK, 1), lambda i: (i, 0)),
            pl.BlockSpec((1, 1), lambda i: (0, 0)),
        ],
        out_shape=(
            jax.ShapeDtypeStruct((_N, _W), jnp.float32),
            jax.ShapeDtypeStruct((_N, _W), jnp.float32),
            jax.ShapeDtypeStruct((_N, 1), jnp.int32),
            jax.ShapeDtypeStruct((1, 1), jnp.float32),
        ),
        scratch_shapes=[pltpu.VMEM((1, 1), jnp.float32)],
        compiler_params=pltpu.CompilerParams(
            dimension_semantics=("arbitrary",),
            vmem_limit_bytes=64 * 1024 * 1024,
        ),
    )(x, labels, w1b, b1, w2b, b2)


_GDN = lax.GatherDimensionNumbers(offset_dims=(), collapsed_slice_dims=(0,),
                                  start_index_map=(0,))


def _take(v, idx):
    return lax.gather(v, idx[:, None], _GDN, (1,),
                      mode=lax.GatherScatterMode.PROMISE_IN_BOUNDS)


def _lane():
    return lax.iota(jnp.int32, _L)


def _prefix_incl(v):
    lane = _lane()
    for k in (1, 2, 4, 8):
        idx = jnp.maximum(lane - k, 0)
        msk = jnp.minimum(jnp.maximum(lane - (k - 1), 0), 1)
        v = v + _take(v, idx) * msk
    return v


def _splat_last(v):
    return _take(v, _lane() * 0 + (_L - 1))


_SC_MESH = plsc.VectorSubcoreMesh(core_axis_name="c", subcore_axis_name="s")


@functools.partial(
    pl.kernel, mesh=_SC_MESH,
    out_type=[jax.ShapeDtypeStruct((_N + 2 * _BLK, _C), jnp.float32),
              jax.ShapeDtypeStruct((_N + 2 * _BLK, _W), jnp.float32),
              jax.ShapeDtypeStruct((16,), jnp.int32),
              jax.ShapeDtypeStruct((_N,), jnp.int32)],
    scratch_types=[pltpu.VMEM((_N,), jnp.int32),
                   pltpu.VMEM((_N,), jnp.int32),
                   pltpu.VMEM((_N,), jnp.int32),
                   pltpu.VMEM((_CH,), jnp.int32),
                   pltpu.VMEM((_CH, _C), jnp.float32),
                   pltpu.VMEM((_CH, _W), jnp.float32),
                   pltpu.VMEM((16,), jnp.int32),
                   pltpu.SemaphoreType.DMA,
                   pltpu.SemaphoreType.DMA])
def _sc_gather(mask_hbm, x_hbm, side_hbm,
               xg_hbm, sideg_hbm, cnt_hbm, sel_hbm,
               mask_v, addr_v, sel_v, addrw_v, rows_v, srow_v, cnt_v,
               sem, sem2):
    wid = lax.axis_index("s") * 2 + lax.axis_index("c")
    pltpu.sync_copy(mask_hbm, mask_v)
    lane = _lane()

    def comp(i, wpos_v):
        mi = mask_v[pl.ds(i * _L, _L)]
        pre = _prefix_incl(mi)
        tot = _splat_last(pre)
        gpos = wpos_v + pre - mi
        src = lane + i * _L
        addr = gpos * mi + (_N + lane) * (1 - mi)
        addr_v[pl.ds(i * _L, _L)] = addr
        sel_v[pl.ds(i * _L, _L)] = gpos * mi + (_N + src) * (1 - mi)
        return wpos_v + tot

    wpos_v = lax.fori_loop(0, _N // _L, comp, jnp.zeros((_L,), jnp.int32))

    @pl.when(wid == 0)
    def _():
        cnt_v[:] = wpos_v
        pltpu.sync_copy(cnt_v, cnt_hbm)
        pltpu.sync_copy(sel_v, sel_hbm)

    for c in range(_RPW // _CH):
        base = wid * _RPW + c * _CH
        av = addr_v[pl.ds(base, _CH)]
        ist = jnp.minimum(jnp.maximum(av - (_N - 1), 0), 1)
        av = av * (1 - ist) + (_N + wid * _CH + lane) * ist
        addrw_v[pl.ds(0, _CH)] = av
        pltpu.sync_copy(x_hbm.at[pl.ds(base, _CH)], rows_v)
        pltpu.async_copy(rows_v, xg_hbm.at[addrw_v], sem).wait()
        pltpu.sync_copy(side_hbm.at[pl.ds(base, _CH)], srow_v)
        pltpu.async_copy(srow_v, sideg_hbm.at[addrw_v], sem2).wait()


def _heads_kernel(cnt_ref, xg_ref, sideg_ref, w1_ref, b1_ref, w2_ref, b2_ref,
                  zg_ref, cls_ref, acc_cls):
    j = pl.program_id(0)
    cnt = cnt_ref[0]

    @pl.when(j == 0)
    def _init():
        acc_cls[:, :] = jnp.zeros_like(acc_cls)

    @pl.when(j * _BLK < cnt)
    def _active():
        xb = xg_ref[:, :].astype(jnp.bfloat16)
        rowid = j * _BLK + lax.broadcasted_iota(jnp.int32, (_BLK, 1), 0)
        lmask = rowid < cnt
        hbs = []
        for c in range(_NC - 1):
            h = jax.lax.dot_general(xb, w1_ref[c], (((1,), (1,)), ((), ())),
                                    preferred_element_type=jnp.float32)
            h = jnp.maximum(h + b1_ref[c], 0.0)
            hbs.append(h.astype(jnp.bfloat16))
        cols = [sideg_ref[:, 7:8]]
        part = jnp.zeros((1, 1), jnp.float32)
        for c in range(_NC - 1):
            z = jax.lax.dot_general(hbs[c], w2_ref[c], (((1,), (0,)), ((), ())),
                                    preferred_element_type=jnp.float32)
            z = z + b2_ref[c, 0]
            y = sideg_ref[:, c:c + 1]
            part = part + jnp.sum(jnp.where(lmask, _bce(z, y), 0.0),
                                  axis=(0, 1), keepdims=True)
            cols.append(jax.nn.sigmoid(z))
        acc_cls[:, :] += part
        zg_ref[:, :] = jnp.concatenate(
            cols + [jnp.zeros((_BLK, _W - 8), jnp.float32)], axis=1)

    @pl.when(j == _GRID - 1)
    def _fin():
        cls_ref[:, :] = acc_cls[:, :] / jnp.maximum(cnt.astype(jnp.float32), 1.0)


def _run_heads(cnt, xg, sideg, w1, b1, w2, b2):
    full = lambda *shape: pl.BlockSpec(shape, lambda i, c: (0,) * len(shape))
    grid_spec = pltpu.PrefetchScalarGridSpec(
        num_scalar_prefetch=1,
        grid=(_GRID,),
        in_specs=[
            pl.BlockSpec((_BLK, _C), lambda i, c: (i, 0)),
            pl.BlockSpec((_BLK, _W), lambda i, c: (i, 0)),
            full(_NC - 1, _H, _C),
            full(_NC - 1, 1, _H),
            full(_NC - 1, _H, 1),
            full(_NC - 1, 1),
        ],
        out_specs=[
            pl.BlockSpec((_BLK, _W), lambda i, c: (i, 0)),
            pl.BlockSpec((1, 1), lambda i, c: (0, 0)),
        ],
        scratch_shapes=[pltpu.VMEM((1, 1), jnp.float32)],
    )
    return pl.pallas_call(
        _heads_kernel,
        grid_spec=grid_spec,
        out_shape=(
            jax.ShapeDtypeStruct((_N, _W), jnp.float32),
            jax.ShapeDtypeStruct((1, 1), jnp.float32),
        ),
        compiler_params=pltpu.CompilerParams(
            dimension_semantics=("arbitrary",),
            vmem_limit_bytes=64 * 1024 * 1024,
        ),
    )(cnt, xg, sideg, w1, b1, w2, b2)


@functools.partial(
    pl.kernel, mesh=_SC_MESH,
    out_type=jax.ShapeDtypeStruct((_N, _W), jnp.float32),
    scratch_types=[pltpu.VMEM((_CHM,), jnp.int32),
                   pltpu.VMEM((_CHM, _W), jnp.float32),
                   pltpu.SemaphoreType.DMA])
def _sc_merge(sel_hbm, src_hbm, out_hbm, selw_v, rows_v, sem):
    wid = lax.axis_index("s") * 2 + lax.axis_index("c")
    for c in range(_RPW // _CHM):
        base = wid * _RPW + c * _CHM
        pltpu.sync_copy(sel_hbm.at[pl.ds(base, _CHM)], selw_v)
        pltpu.async_copy(src_hbm.at[selw_v], rows_v, sem).wait()
        pltpu.sync_copy(rows_v, out_hbm.at[pl.ds(base, _CHM)])


def kernel(fuse_embeddings, segment_classes, pn_W1, pn_b1, pn_W2, pn_b2,
           cat_W1, cat_b1, cat_W2, cat_b2):
    seg = segment_classes.reshape(-1).astype(jnp.int32)
    lab0 = (seg > 0).astype(jnp.float32)
    labc = (seg[:, None] == jnp.arange(1, _NC, dtype=jnp.int32)[None, :]).astype(jnp.float32)
    labels = jnp.concatenate([lab0[:, None], labc], axis=1)

    x = fuse_embeddings.reshape(_N, _C)

    cp16, side, maskN1, pn = _run_head0(
        x, labels, pn_W1.astype(jnp.bfloat16), pn_b1.reshape(1, _H),
        pn_W2.astype(jnp.bfloat16).reshape(_H, 1), pn_b2.reshape(1, 1))

    xg, sideg, cnt16, sel = _sc_gather(maskN1.reshape(_N), x, side)

    zg, cls = _run_heads(
        cnt16[0:1], xg, sideg,
        cat_W1.astype(jnp.bfloat16),
        cat_b1.reshape(_NC - 1, 1, _H),
        cat_W2.astype(jnp.bfloat16).reshape(_NC - 1, _H, 1),
        cat_b2.reshape(_NC - 1, 1))

    src = jnp.concatenate([zg, cp16], axis=0)
    cpf = _sc_merge(sel, src)

    return pn[0, 0], cls.reshape(1), cpf[:, :8]


# speedup vs baseline: 65.5271x; 1.0356x over previous
import functools

import jax
import jax.numpy as jnp
from jax import lax
from jax.experimental import pallas as pl
from jax.experimental.pallas import tpu as pltpu
from jax.experimental.pallas import tpu_sc as plsc

_NC = 8
_C = 2048
_H = _C // 2
_N = 8192
_BLK = 256
_GRID = _N // _BLK

_L = 16
_NW = 32
_RPW = _N // _NW
_CH = 16
_CHM = 128
_W = 128


def _bce(z, y):
    return jnp.maximum(z, 0.0) - z * y + jnp.log1p(jnp.exp(-jnp.abs(z)))


def _head0_kernel(x_ref, labels_ref, w1_ref, b1_ref, w2_ref, b2_ref,
                  cp_ref, side_ref, mask_ref, pn_ref, acc_pn):
    i = pl.program_id(0)

    @pl.when(i == 0)
    def _init():
        acc_pn[:, :] = jnp.zeros_like(acc_pn)

    xb = x_ref[:, :].astype(jnp.bfloat16)
    h = jax.lax.dot_general(xb, w1_ref[:, :], (((1,), (1,)), ((), ())),
                            preferred_element_type=jnp.float32)
    h = jnp.maximum(h + b1_ref[:, :], 0.0)
    z = jax.lax.dot_general(h.astype(jnp.bfloat16), w2_ref[:, :],
                            (((1,), (0,)), ((), ())),
                            preferred_element_type=jnp.float32)
    z = z + b2_ref[0, 0]
    y0 = labels_ref[:, 0:1]
    acc_pn[:, :] += jnp.sum(_bce(z, y0), axis=(0, 1), keepdims=True)
    probs = jax.nn.sigmoid(z)
    mask = probs >= 0.5
    cp_ref[:, :] = jnp.concatenate(
        [probs, jnp.zeros((_BLK, _W - 1), jnp.float32)], axis=1)
    side_ref[:, :] = jnp.concatenate(
        [labels_ref[:, 1:8], probs, jnp.zeros((_BLK, _W - 8), jnp.float32)],
        axis=1)
    mask_ref[:, :] = mask.astype(jnp.int32)

    @pl.when(i == _GRID - 1)
    def _fin():
        pn_ref[:, :] = acc_pn[:, :] * (1.0 / _N)


def _run_head0(x, labels, w1b, b1, w2b, b2):
    full = lambda *shape: pl.BlockSpec(shape, lambda i: (0,) * len(shape))
    return pl.pallas_call(
        _head0_kernel,
        grid=(_GRID,),
        in_specs=[
            pl.BlockSpec((_BLK, _C), lambda i: (i, 0)),
            pl.BlockSpec((_BLK, _NC), lambda i: (i, 0)),
            full(_H, _C),
            full(1, _H),
            full(_H, 1),
            full(1, 1),
        ],
        out_specs=[
            pl.BlockSpec((_BLK, _W), lambda i: (i, 0)),
            pl.BlockSpec((_BLK, _W), lambda i: (i, 0)),
            pl.BlockSpec((_BLK, 1), lambda i: (i, 0)),
            pl.BlockSpec((1, 1), lambda i: (0, 0)),
        ],
        out_shape=(
            jax.ShapeDtypeStruct((_N, _W), jnp.float32),
            jax.ShapeDtypeStruct((_N, _W), jnp.float32),
            jax.ShapeDtypeStruct((_N, 1), jnp.int32),
            jax.ShapeDtypeStruct((1, 1), jnp.float32),
        ),
        scratch_shapes=[pltpu.VMEM((1, 1), jnp.float32)],
        compiler_params=pltpu.CompilerParams(
            dimension_semantics=("arbitrary",),
            vmem_limit_bytes=64 * 1024 * 1024,
        ),
    )(x, labels, w1b, b1, w2b, b2)


_GDN = lax.GatherDimensionNumbers(offset_dims=(), collapsed_slice_dims=(0,),
                                  start_index_map=(0,))


def _take(v, idx):
    return lax.gather(v, idx[:, None], _GDN, (1,),
                      mode=lax.GatherScatterMode.PROMISE_IN_BOUNDS)


def _lane():
    return lax.iota(jnp.int32, _L)


def _prefix_incl(v):
    lane = _lane()
    for k in (1, 2, 4, 8):
        idx = jnp.maximum(lane - k, 0)
        msk = jnp.minimum(jnp.maximum(lane - (k - 1), 0), 1)
        v = v + _take(v, idx) * msk
    return v


def _splat_last(v):
    return _take(v, _lane() * 0 + (_L - 1))


_SC_MESH = plsc.VectorSubcoreMesh(core_axis_name="c", subcore_axis_name="s")


@functools.partial(
    pl.kernel, mesh=_SC_MESH,
    out_type=[jax.ShapeDtypeStruct((_N + 2 * _BLK, _C), jnp.float32),
              jax.ShapeDtypeStruct((_N + 2 * _BLK, _W), jnp.float32),
              jax.ShapeDtypeStruct((16,), jnp.int32),
              jax.ShapeDtypeStruct((_N,), jnp.int32)],
    scratch_types=[pltpu.VMEM((_N,), jnp.int32),
                   pltpu.VMEM((_N,), jnp.int32),
                   pltpu.VMEM((_N,), jnp.int32),
                   pltpu.VMEM((_CH,), jnp.int32),
                   pltpu.VMEM((_CH, _C), jnp.float32),
                   pltpu.VMEM((_CH, _W), jnp.float32),
                   pltpu.VMEM((16,), jnp.int32),
                   pltpu.SemaphoreType.DMA,
                   pltpu.SemaphoreType.DMA])
def _sc_gather(mask_hbm, x_hbm, side_hbm,
               xg_hbm, sideg_hbm, cnt_hbm, sel_hbm,
               mask_v, addr_v, sel_v, addrw_v, rows_v, srow_v, cnt_v,
               sem, sem2):
    wid = lax.axis_index("s") * 2 + lax.axis_index("c")
    pltpu.sync_copy(mask_hbm, mask_v)
    lane = _lane()

    def comp(i, wpos_v):
        mi = mask_v[pl.ds(i * _L, _L)]
        pre = _prefix_incl(mi)
        tot = _splat_last(pre)
        gpos = wpos_v + pre - mi
        src = lane + i * _L
        addr = gpos * mi + (_N + lane) * (1 - mi)
        addr_v[pl.ds(i * _L, _L)] = addr
        sel_v[pl.ds(i * _L, _L)] = gpos * mi + (_N + src) * (1 - mi)
        return wpos_v + tot

    wpos_v = lax.fori_loop(0, _N // _L, comp, jnp.zeros((_L,), jnp.int32))

    @pl.when(wid == 0)
    def _():
        cnt_v[:] = wpos_v
        pltpu.sync_copy(cnt_v, cnt_hbm)
        pltpu.sync_copy(sel_v, sel_hbm)

    for c in range(_RPW // _CH):
        base = wid * _RPW + c * _CH
        ra = pltpu.async_copy(x_hbm.at[pl.ds(base, _CH)], rows_v, sem)
        rb = pltpu.async_copy(side_hbm.at[pl.ds(base, _CH)], srow_v, sem2)
        av = addr_v[pl.ds(base, _CH)]
        ist = jnp.minimum(jnp.maximum(av - (_N - 1), 0), 1)
        av = av * (1 - ist) + (_N + wid * _CH + lane) * ist
        addrw_v[pl.ds(0, _CH)] = av
        ra.wait()
        rb.wait()
        sa = pltpu.async_copy(rows_v, xg_hbm.at[addrw_v], sem)
        sb = pltpu.async_copy(srow_v, sideg_hbm.at[addrw_v], sem2)
        sa.wait()
        sb.wait()


def _heads_kernel(cnt_ref, xg_ref, sideg_ref, w1_ref, b1_ref, w2_ref, b2_ref,
                  zg_ref, cls_ref, acc_cls):
    j = pl.program_id(0)
    cnt = cnt_ref[0]

    @pl.when(j == 0)
    def _init():
        acc_cls[:, :] = jnp.zeros_like(acc_cls)

    @pl.when(j * _BLK < cnt)
    def _active():
        xb = xg_ref[:, :].astype(jnp.bfloat16)
        rowid = j * _BLK + lax.broadcasted_iota(jnp.int32, (_BLK, 1), 0)
        lmask = rowid < cnt
        hbs = []
        for c in range(_NC - 1):
            h = jax.lax.dot_general(xb, w1_ref[c], (((1,), (1,)), ((), ())),
                                    preferred_element_type=jnp.float32)
            h = jnp.maximum(h + b1_ref[c], 0.0)
            hbs.append(h.astype(jnp.bfloat16))
        cols = [sideg_ref[:, 7:8]]
        part = jnp.zeros((1, 1), jnp.float32)
        for c in range(_NC - 1):
            z = jax.lax.dot_general(hbs[c], w2_ref[c], (((1,), (0,)), ((), ())),
                                    preferred_element_type=jnp.float32)
            z = z + b2_ref[c, 0]
            y = sideg_ref[:, c:c + 1]
            part = part + jnp.sum(jnp.where(lmask, _bce(z, y), 0.0),
                                  axis=(0, 1), keepdims=True)
            cols.append(jax.nn.sigmoid(z))
        acc_cls[:, :] += part
        zg_ref[:, :] = jnp.concatenate(
            cols + [jnp.zeros((_BLK, _W - 8), jnp.float32)], axis=1)

    @pl.when(j == _GRID - 1)
    def _fin():
        cls_ref[:, :] = acc_cls[:, :] / jnp.maximum(cnt.astype(jnp.float32), 1.0)


def _run_heads(cnt, xg, sideg, w1, b1, w2, b2):
    full = lambda *shape: pl.BlockSpec(shape, lambda i, c: (0,) * len(shape))
    grid_spec = pltpu.PrefetchScalarGridSpec(
        num_scalar_prefetch=1,
        grid=(_GRID,),
        in_specs=[
            pl.BlockSpec((_BLK, _C), lambda i, c: (i, 0)),
            pl.BlockSpec((_BLK, _W), lambda i, c: (i, 0)),
            full(_NC - 1, _H, _C),
            full(_NC - 1, 1, _H),
            full(_NC - 1, _H, 1),
            full(_NC - 1, 1),
        ],
        out_specs=[
            pl.BlockSpec((_BLK, _W), lambda i, c: (i, 0)),
            pl.BlockSpec((1, 1), lambda i, c: (0, 0)),
        ],
        scratch_shapes=[pltpu.VMEM((1, 1), jnp.float32)],
    )
    return pl.pallas_call(
        _heads_kernel,
        grid_spec=grid_spec,
        out_shape=(
            jax.ShapeDtypeStruct((_N, _W), jnp.float32),
            jax.ShapeDtypeStruct((1, 1), jnp.float32),
        ),
        compiler_params=pltpu.CompilerParams(
            dimension_semantics=("arbitrary",),
            vmem_limit_bytes=64 * 1024 * 1024,
        ),
    )(cnt, xg, sideg, w1, b1, w2, b2)


@functools.partial(
    pl.kernel, mesh=_SC_MESH,
    out_type=jax.ShapeDtypeStruct((_N, _W), jnp.float32),
    scratch_types=[pltpu.VMEM((_CHM,), jnp.int32),
                   pltpu.VMEM((_CHM, _W), jnp.float32),
                   pltpu.SemaphoreType.DMA])
def _sc_merge(sel_hbm, src_hbm, out_hbm, selw_v, rows_v, sem):
    wid = lax.axis_index("s") * 2 + lax.axis_index("c")
    for c in range(_RPW // _CHM):
        base = wid * _RPW + c * _CHM
        pltpu.sync_copy(sel_hbm.at[pl.ds(base, _CHM)], selw_v)
        pltpu.async_copy(src_hbm.at[selw_v], rows_v, sem).wait()
        pltpu.sync_copy(rows_v, out_hbm.at[pl.ds(base, _CHM)])


def kernel(fuse_embeddings, segment_classes, pn_W1, pn_b1, pn_W2, pn_b2,
           cat_W1, cat_b1, cat_W2, cat_b2):
    seg = segment_classes.reshape(-1).astype(jnp.int32)
    lab0 = (seg > 0).astype(jnp.float32)
    labc = (seg[:, None] == jnp.arange(1, _NC, dtype=jnp.int32)[None, :]).astype(jnp.float32)
    labels = jnp.concatenate([lab0[:, None], labc], axis=1)

    x = fuse_embeddings.reshape(_N, _C)

    cp16, side, maskN1, pn = _run_head0(
        x, labels, pn_W1.astype(jnp.bfloat16), pn_b1.reshape(1, _H),
        pn_W2.astype(jnp.bfloat16).reshape(_H, 1), pn_b2.reshape(1, 1))

    xg, sideg, cnt16, sel = _sc_gather(maskN1.reshape(_N), x, side)

    zg, cls = _run_heads(
        cnt16[0:1], xg, sideg,
        cat_W1.astype(jnp.bfloat16),
        cat_b1.reshape(_NC - 1, 1, _H),
        cat_W2.astype(jnp.bfloat16).reshape(_NC - 1, _H, 1),
        cat_b2.reshape(_NC - 1, 1))

    src = jnp.concatenate([zg, cp16], axis=0)
    cpf = _sc_merge(sel, src)

    return pn[0, 0], cls.reshape(1), cpf[:, :8]
